# baseline (speedup 1.0000x reference)
import functools

import jax
import jax.numpy as jnp
from jax import lax
from jax.experimental import pallas as pl
from jax.experimental.pallas import tpu as pltpu
from jax.experimental.pallas import tpu_sc as plsc

N_CORES = 2
N_SUBCORES = 16
N_WORKERS = N_CORES * N_SUBCORES
WINDOW = 800
GRID = 4


def _sc_segment_sum16(table, edges, zeros):
  npad, f = table.shape
  ne = edges.shape[1]
  ept = ne // N_WORKERS
  nwin = ept // WINDOW
  npairs = nwin // 2
  rpt = npad // N_SUBCORES

  mesh = plsc.VectorSubcoreMesh(core_axis_name="c", subcore_axis_name="s")

  @functools.partial(
      pl.kernel,
      mesh=mesh,
      out_type=jax.ShapeDtypeStruct((N_CORES, npad, f), jnp.float32),
      scratch_types=[
          pltpu.VMEM_SHARED((npad, f), jnp.float32),
          pltpu.VMEM((WINDOW,), jnp.int32),
          pltpu.VMEM((WINDOW,), jnp.int32),
          pltpu.VMEM((WINDOW,), jnp.int32),
          pltpu.VMEM((WINDOW,), jnp.int32),
          pltpu.VMEM((WINDOW, f), jnp.float32),
          pltpu.VMEM((WINDOW, f), jnp.float32),
          pltpu.SemaphoreType.DMA,
          pltpu.SemaphoreType.DMA,
          pltpu.SemaphoreType.DMA,
          pltpu.SemaphoreType.DMA,
      ],
      compiler_params=pltpu.CompilerParams(use_tc_tiling_on_sc=False),
  )
  def seg(table_hbm, edge_hbm, z_hbm, out_hbm, agg_sh,
          is0, id0, is1, id1, r0v, r1v, semi0, semi1, semg0, semg1):
    c = lax.axis_index("c")
    s = lax.axis_index("s")
    row0 = s * rpt
    pltpu.sync_copy(z_hbm.at[pl.ds(row0, rpt)], agg_sh.at[pl.ds(row0, rpt)])
    plsc.subcore_barrier()

    ebase = (c * N_SUBCORES + s) * ept
    idx = [(is0, id0, semi0), (is1, id1, semi1)]
    rows = [(r0v, semg0), (r1v, semg1)]

    def load_idx(w, slot):
      i_s, i_d, sem = idx[slot]
      pltpu.async_copy(edge_hbm.at[0, pl.ds(ebase + w * WINDOW, WINDOW)],
                       i_s, sem)
      pltpu.async_copy(edge_hbm.at[1, pl.ds(ebase + w * WINDOW, WINDOW)],
                       i_d, sem)

    def wait_idx(slot):
      i_s, i_d, sem = idx[slot]
      pltpu.make_async_copy(edge_hbm.at[0, pl.ds(0, WINDOW)], i_s, sem).wait()
      pltpu.make_async_copy(edge_hbm.at[1, pl.ds(0, WINDOW)], i_d, sem).wait()

    def gather(slot):
      i_s = idx[slot][0]
      rv, sem = rows[slot]
      pltpu.async_copy(table_hbm.at[i_s], rv, sem)

    def wait_gather(slot):
      i_s = idx[slot][0]
      rv, sem = rows[slot]
      pltpu.make_async_copy(table_hbm.at[i_s], rv, sem).wait()

    def scatter(slot):
      i_d = idx[slot][1]
      rv = rows[slot][0]
      pltpu.sync_copy(rv, agg_sh.at[i_d], add=True)

    load_idx(0, 0)
    wait_idx(0)
    gather(0)
    load_idx(1, 1)

    def body(k, carry):
      w = 2 * k
      wait_idx(1)
      gather(1)
      wait_gather(0)
      scatter(0)

      @pl.when(w + 2 < nwin)
      def _():
        load_idx(w + 2, 0)
        wait_idx(0)
        gather(0)
      wait_gather(1)
      scatter(1)

      @pl.when(w + 3 < nwin)
      def _():
        load_idx(w + 3, 1)
      return carry

    lax.fori_loop(0, npairs, body, 0)
    if nwin % 2:
      wait_gather(0)
      scatter(0)
    plsc.subcore_barrier()
    pltpu.sync_copy(agg_sh.at[pl.ds(row0, rpt)],
                    out_hbm.at[c, pl.ds(row0, rpt)])

  return seg(table, edges, zeros)


def _tc_project_x(xg, w_rel_big, w_root_big, b1_tiled):
  nrows = xg.shape[0]

  def body(x_ref, wr_ref, wt_ref, b_ref, xr_ref, hp_ref):
    x = x_ref[...]
    xr_ref[...] = jnp.dot(x, wr_ref[...].T,
                          preferred_element_type=jnp.float32, precision=lax.Precision.HIGHEST)
    hp_ref[...] = (jnp.dot(x, wt_ref[...].T,
                           preferred_element_type=jnp.float32, precision=lax.Precision.HIGHEST) + b_ref[...])

  return pl.pallas_call(
      body,
      out_shape=[
          jax.ShapeDtypeStruct((nrows, 1024), jnp.float32),
          jax.ShapeDtypeStruct((nrows, 1024), jnp.float32),
      ],
  )(xg, w_rel_big, w_root_big, b1_tiled)


def _tc_combine1(agg1g, hprefg):
  nrows = hprefg.shape[0]
  blk = nrows // GRID

  def body(agg_ref, hp_ref, out_ref):
    z = agg_ref[0] + agg_ref[1] + hp_ref[...]
    out_ref[...] = jnp.where(z > 0.0, z, jnp.exp(z) - 1.0)

  return pl.pallas_call(
      body,
      grid=(GRID,),
      in_specs=[
          pl.BlockSpec((2, blk, 128), lambda i: (0, i, 0)),
          pl.BlockSpec((blk, 128), lambda i: (i, 0)),
      ],
      out_specs=pl.BlockSpec((blk, 128), lambda i: (i, 0)),
      out_shape=jax.ShapeDtypeStruct((nrows, 128), jnp.float32),
  )(agg1g, hprefg)


def _tc_layer2_head(agg2g, hg, nrows_valid, w2_rel_big, b2_tiled,
                    w2_root_big, wl1, bl1, wl2, bl2, wl3, bl3):
  nrows = hg.shape[0]
  blk = nrows // GRID

  def body(agg_ref, h_ref, wr_ref, b_ref, wt_ref,
           l1_ref, c1_ref, l2_ref, c2_ref, l3_ref, c3_ref,
           out_ref, gmax_ref):
    i = pl.program_id(0)
    agg = agg_ref[0] + agg_ref[1]
    h2 = (jnp.dot(agg, wr_ref[...].T, preferred_element_type=jnp.float32, precision=lax.Precision.HIGHEST)
          + jnp.dot(h_ref[...], wt_ref[...].T,
                    preferred_element_type=jnp.float32, precision=lax.Precision.HIGHEST)
          + b_ref[...])
    row = lax.broadcasted_iota(jnp.int32, h2.shape, 0) + i * blk
    h2 = jnp.where(row < nrows_valid, h2, -jnp.inf)
    bm = jnp.max(h2, axis=0, keepdims=True)

    @pl.when(i == 0)
    def _():
      gmax_ref[...] = bm

    @pl.when(i > 0)
    def _():
      gmax_ref[...] = jnp.maximum(gmax_ref[...], bm)

    @pl.when(i == GRID - 1)
    def _():
      gm = gmax_ref[...]
      g = gm[:, 0:32]
      for j in range(1, 8):
        g = jnp.maximum(g, gm[:, 32 * j:32 * j + 32])
      g = jnp.maximum(
          jnp.dot(g, l1_ref[...].T, preferred_element_type=jnp.float32, precision=lax.Precision.HIGHEST)
          + c1_ref[...], 0.0)
      g = jnp.maximum(
          jnp.dot(g, l2_ref[...].T, preferred_element_type=jnp.float32, precision=lax.Precision.HIGHEST)
          + c2_ref[...], 0.0)
      out_ref[...] = (
          jnp.sum(g * l3_ref[...], axis=1, keepdims=True) + c3_ref[...])

  full = lambda s: pl.BlockSpec(s, lambda i: (0,) * len(s))
  return pl.pallas_call(
      body,
      grid=(GRID,),
      in_specs=[
          pl.BlockSpec((2, blk, 128), lambda i: (0, i, 0)),
          pl.BlockSpec((blk, 128), lambda i: (i, 0)),
          full((256, 128)), full((1, 256)), full((256, 128)),
          full((16, 32)), full((1, 16)),
          full((8, 16)), full((1, 8)),
          full((1, 8)), full((1, 1)),
      ],
      out_specs=pl.BlockSpec((1, 1), lambda i: (0, 0)),
      out_shape=jax.ShapeDtypeStruct((1, 1), jnp.float32),
      scratch_shapes=[pltpu.VMEM((1, 256), jnp.float32)],
  )(agg2g, hg, w2_rel_big, b2_tiled, w2_root_big,
    wl1, bl1, wl2, bl2, wl3, bl3)


def kernel(x, edge_index, W1_rel, b1, W1_root, W2_rel, b2, W2_root,
           Wl1, bl1, Wl2, bl2, Wl3, bl3):
  n = x.shape[0]
  npad = ((n + 127) // 128) * 128
  edges = edge_index.astype(jnp.int32)
  xp = jnp.pad(x.astype(jnp.float32), ((0, npad - n), (0, 0)))
  xg = xp.reshape(npad // 64, 128)
  z16 = jnp.zeros((npad, 16), jnp.float32)

  eye64 = jnp.eye(64, dtype=jnp.float32)
  eye8 = jnp.eye(8, dtype=jnp.float32)
  w1_rel_big = jnp.kron(eye64, W1_rel)
  w1_root_big = jnp.kron(eye64, W1_root)
  b1_tiled = jnp.tile(b1, 64).reshape(1, 1024)
  w2_rel_big = jnp.kron(eye8, W2_rel)
  w2_root_big = jnp.kron(eye8, W2_root)
  b2_tiled = jnp.tile(b2, 8).reshape(1, 256)

  xr_g, hpref_g = _tc_project_x(xg, w1_rel_big, w1_root_big, b1_tiled)
  xr = xr_g.reshape(npad, 16)
  agg1r = _sc_segment_sum16(xr, edges, z16)
  hg = _tc_combine1(agg1r.reshape(2, npad // 8, 128),
                    hpref_g.reshape(npad // 8, 128))
  h = hg.reshape(npad, 16)
  agg2 = _sc_segment_sum16(h, edges, z16)
  out = _tc_layer2_head(
      agg2.reshape(2, npad // 8, 128), hg, n // 8,
      w2_rel_big, b2_tiled, w2_root_big,
      Wl1, bl1.reshape(1, -1), Wl2, bl2.reshape(1, -1),
      Wl3, bl3.reshape(1, -1))
  return out


# speedup vs baseline: 1.1884x; 1.1884x over previous
import functools

import jax
import jax.numpy as jnp
from jax import lax
from jax.experimental import pallas as pl
from jax.experimental.pallas import tpu as pltpu
from jax.experimental.pallas import tpu_sc as plsc

N_CORES = 2
N_SUBCORES = 16
N_WORKERS = N_CORES * N_SUBCORES
WINDOW = 800
GRID = 4


def _sc_segment_sum16(table, edges, zeros):
  npad, f = table.shape
  ne = edges.shape[1]
  ept = ne // N_WORKERS
  nwin = ept // WINDOW
  npairs = nwin // 2
  rpt = npad // N_SUBCORES

  mesh = plsc.VectorSubcoreMesh(core_axis_name="c", subcore_axis_name="s")

  @functools.partial(
      pl.kernel,
      mesh=mesh,
      out_type=jax.ShapeDtypeStruct((N_CORES, npad, f), jnp.float32),
      scratch_types=[
          pltpu.VMEM_SHARED((npad, f), jnp.float32),
          pltpu.VMEM((3, 2, WINDOW), jnp.int32),
          pltpu.VMEM((WINDOW, f), jnp.float32),
          pltpu.VMEM((WINDOW, f), jnp.float32),
          pltpu.SemaphoreType.DMA((3,)),
          pltpu.SemaphoreType.DMA,
          pltpu.SemaphoreType.DMA,
      ],
      compiler_params=pltpu.CompilerParams(use_tc_tiling_on_sc=False),
  )
  def seg(table_hbm, edge_hbm, z_hbm, out_hbm, agg_sh,
          idx3, r0v, r1v, sem3, semg0, semg1):
    c = lax.axis_index("c")
    s = lax.axis_index("s")
    row0 = s * rpt
    pltpu.sync_copy(z_hbm.at[pl.ds(row0, rpt)], agg_sh.at[pl.ds(row0, rpt)])
    plsc.subcore_barrier()

    ebase = (c * N_SUBCORES + s) * ept
    rows = [(r0v, semg0), (r1v, semg1)]

    def load_idx(w):
      m = lax.rem(w, 3) if not isinstance(w, int) else w % 3
      pltpu.async_copy(edge_hbm.at[0, pl.ds(ebase + w * WINDOW, WINDOW)],
                       idx3.at[m, 0], sem3.at[m])
      pltpu.async_copy(edge_hbm.at[1, pl.ds(ebase + w * WINDOW, WINDOW)],
                       idx3.at[m, 1], sem3.at[m])

    def wait_idx(w):
      m = lax.rem(w, 3) if not isinstance(w, int) else w % 3
      pltpu.make_async_copy(edge_hbm.at[0, pl.ds(0, WINDOW)],
                            idx3.at[m, 0], sem3.at[m]).wait()
      pltpu.make_async_copy(edge_hbm.at[1, pl.ds(0, WINDOW)],
                            idx3.at[m, 1], sem3.at[m]).wait()

    def gather(w, rslot):
      m = lax.rem(w, 3) if not isinstance(w, int) else w % 3
      rv, sem = rows[rslot]
      pltpu.async_copy(table_hbm.at[idx3.at[m, 0]], rv, sem)

    def wait_gather(w, rslot):
      m = lax.rem(w, 3) if not isinstance(w, int) else w % 3
      rv, sem = rows[rslot]
      pltpu.make_async_copy(table_hbm.at[idx3.at[m, 0]], rv, sem).wait()

    def scatter(w, rslot):
      m = lax.rem(w, 3) if not isinstance(w, int) else w % 3
      rv = rows[rslot][0]
      pltpu.sync_copy(rv, agg_sh.at[idx3.at[m, 1]], add=True)

    load_idx(0)
    load_idx(1)
    load_idx(2)
    wait_idx(0)
    gather(0, 0)

    def body(k, carry):
      w = 2 * k
      wait_idx(w + 1)
      gather(w + 1, 1)
      wait_gather(w, 0)
      scatter(w, 0)

      @pl.when(w + 3 < nwin)
      def _():
        load_idx(w + 3)

      @pl.when(w + 2 < nwin)
      def _():
        wait_idx(w + 2)
        gather(w + 2, 0)
      wait_gather(w + 1, 1)
      scatter(w + 1, 1)

      @pl.when(w + 4 < nwin)
      def _():
        load_idx(w + 4)
      return carry

    lax.fori_loop(0, npairs, body, 0)
    if nwin % 2:
      wait_gather(nwin - 1, 0)
      scatter(nwin - 1, 0)
    plsc.subcore_barrier()
    pltpu.sync_copy(agg_sh.at[pl.ds(row0, rpt)],
                    out_hbm.at[c, pl.ds(row0, rpt)])

  return seg(table, edges, zeros)


def _tc_project_x(xg, w_rel_big, w_root_big, b1_tiled):
  nrows = xg.shape[0]

  def body(x_ref, wr_ref, wt_ref, b_ref, xr_ref, hp_ref):
    x = x_ref[...]
    xr_ref[...] = jnp.dot(x, wr_ref[...].T,
                          preferred_element_type=jnp.float32)
    hp_ref[...] = (jnp.dot(x, wt_ref[...].T,
                           preferred_element_type=jnp.float32) + b_ref[...])

  return pl.pallas_call(
      body,
      out_shape=[
          jax.ShapeDtypeStruct((nrows, 1024), jnp.float32),
          jax.ShapeDtypeStruct((nrows, 1024), jnp.float32),
      ],
  )(xg, w_rel_big, w_root_big, b1_tiled)


def _tc_combine1(agg1g, hprefg):
  nrows = hprefg.shape[0]
  blk = nrows // GRID

  def body(agg_ref, hp_ref, out_ref):
    z = agg_ref[0] + agg_ref[1] + hp_ref[...]
    out_ref[...] = jnp.where(z > 0.0, z, jnp.exp(z) - 1.0)

  return pl.pallas_call(
      body,
      grid=(GRID,),
      in_specs=[
          pl.BlockSpec((2, blk, 128), lambda i: (0, i, 0)),
          pl.BlockSpec((blk, 128), lambda i: (i, 0)),
      ],
      out_specs=pl.BlockSpec((blk, 128), lambda i: (i, 0)),
      out_shape=jax.ShapeDtypeStruct((nrows, 128), jnp.float32),
  )(agg1g, hprefg)


def _tc_layer2_head(agg2g, hg, nrows_valid, w2_rel_big, b2_tiled,
                    w2_root_big, wl1, bl1, wl2, bl2, wl3, bl3):
  nrows = hg.shape[0]
  blk = nrows // GRID

  def body(agg_ref, h_ref, wr_ref, b_ref, wt_ref,
           l1_ref, c1_ref, l2_ref, c2_ref, l3_ref, c3_ref,
           out_ref, gmax_ref):
    i = pl.program_id(0)
    agg = agg_ref[0] + agg_ref[1]
    h2 = (jnp.dot(agg, wr_ref[...].T, preferred_element_type=jnp.float32)
          + jnp.dot(h_ref[...], wt_ref[...].T,
                    preferred_element_type=jnp.float32)
          + b_ref[...])
    row = lax.broadcasted_iota(jnp.int32, h2.shape, 0) + i * blk
    h2 = jnp.where(row < nrows_valid, h2, -jnp.inf)
    bm = jnp.max(h2, axis=0, keepdims=True)

    @pl.when(i == 0)
    def _():
      gmax_ref[...] = bm

    @pl.when(i > 0)
    def _():
      gmax_ref[...] = jnp.maximum(gmax_ref[...], bm)

    @pl.when(i == GRID - 1)
    def _():
      gm = gmax_ref[...]
      g = gm[:, 0:32]
      for j in range(1, 8):
        g = jnp.maximum(g, gm[:, 32 * j:32 * j + 32])
      g = jnp.maximum(
          jnp.dot(g, l1_ref[...].T, preferred_element_type=jnp.float32)
          + c1_ref[...], 0.0)
      g = jnp.maximum(
          jnp.dot(g, l2_ref[...].T, preferred_element_type=jnp.float32)
          + c2_ref[...], 0.0)
      out_ref[...] = (
          jnp.sum(g * l3_ref[...], axis=1, keepdims=True) + c3_ref[...])

  full = lambda s: pl.BlockSpec(s, lambda i: (0,) * len(s))
  return pl.pallas_call(
      body,
      grid=(GRID,),
      in_specs=[
          pl.BlockSpec((2, blk, 128), lambda i: (0, i, 0)),
          pl.BlockSpec((blk, 128), lambda i: (i, 0)),
          full((256, 128)), full((1, 256)), full((256, 128)),
          full((16, 32)), full((1, 16)),
          full((8, 16)), full((1, 8)),
          full((1, 8)), full((1, 1)),
      ],
      out_specs=pl.BlockSpec((1, 1), lambda i: (0, 0)),
      out_shape=jax.ShapeDtypeStruct((1, 1), jnp.float32),
      scratch_shapes=[pltpu.VMEM((1, 256), jnp.float32)],
  )(agg2g, hg, w2_rel_big, b2_tiled, w2_root_big,
    wl1, bl1, wl2, bl2, wl3, bl3)


def kernel(x, edge_index, W1_rel, b1, W1_root, W2_rel, b2, W2_root,
           Wl1, bl1, Wl2, bl2, Wl3, bl3):
  n = x.shape[0]
  npad = ((n + 127) // 128) * 128
  edges = edge_index.astype(jnp.int32)
  xp = jnp.pad(x.astype(jnp.float32), ((0, npad - n), (0, 0)))
  xg = xp.reshape(npad // 64, 128)
  z16 = jnp.zeros((npad, 16), jnp.float32)

  eye64 = jnp.eye(64, dtype=jnp.float32)
  eye8 = jnp.eye(8, dtype=jnp.float32)
  w1_rel_big = jnp.kron(eye64, W1_rel)
  w1_root_big = jnp.kron(eye64, W1_root)
  b1_tiled = jnp.tile(b1, 64).reshape(1, 1024)
  w2_rel_big = jnp.kron(eye8, W2_rel)
  w2_root_big = jnp.kron(eye8, W2_root)
  b2_tiled = jnp.tile(b2, 8).reshape(1, 256)

  xr_g, hpref_g = _tc_project_x(xg, w1_rel_big, w1_root_big, b1_tiled)
  xr = xr_g.reshape(npad, 16)
  agg1r = _sc_segment_sum16(xr, edges, z16)
  hg = _tc_combine1(agg1r.reshape(2, npad // 8, 128),
                    hpref_g.reshape(npad // 8, 128))
  h = hg.reshape(npad, 16)
  agg2 = _sc_segment_sum16(h, edges, z16)
  out = _tc_layer2_head(
      agg2.reshape(2, npad // 8, 128), hg, n // 8,
      w2_rel_big, b2_tiled, w2_root_big,
      Wl1, bl1.reshape(1, -1), Wl2, bl2.reshape(1, -1),
      Wl3, bl3.reshape(1, -1))
  return out
